# baseline (speedup 1.0000x reference)
import jax
import jax.numpy as jnp
from jax.experimental import pallas as pl
from jax.experimental.pallas import tpu as pltpu

_B, _S, _D, _E, _R = 4, 2048, 2048, 8, 4
_ALPHA = 16.0
_ER = _E * _R
_TOKENS_PER_BLOCK = 256


def _molora_block(x_ref, w_ref, b_ref, exp_ref, bcat_ref, out_ref):
    y = jnp.dot(x_ref[...], w_ref[...], preferred_element_type=jnp.float32)
    ax = y[:, :_ER]
    logits = y[:, _ER:_ER + _E] + b_ref[...]
    m = jnp.max(logits, axis=-1, keepdims=True)
    e = jnp.exp(logits - m)
    probs = e / jnp.sum(e, axis=-1, keepdims=True)
    probs_er = jnp.dot(probs, exp_ref[...], preferred_element_type=jnp.float32)
    out_ref[...] = jnp.dot(ax * probs_er, bcat_ref[...],
                           preferred_element_type=jnp.float32)


def kernel(x, lora_A, lora_B, router_w, router_b):
    b, s, d = x.shape
    e, _, r = lora_A.shape
    tokens = b * s
    tb = _TOKENS_PER_BLOCK
    grid = tokens // tb

    x2 = x.reshape(tokens, d)
    a_cat = lora_A.transpose(1, 0, 2).reshape(d, e * r)
    w_fused = jnp.concatenate([a_cat, router_w], axis=1)
    b_cat = lora_B.reshape(e * r, d) * (_ALPHA / r)
    expand = jnp.repeat(jnp.eye(e, dtype=jnp.float32), r, axis=1)
    bias = router_b.reshape(1, e)

    out = pl.pallas_call(
        _molora_block,
        grid=(grid,),
        in_specs=[
            pl.BlockSpec((tb, d), lambda i: (i, 0)),
            pl.BlockSpec((d, e * r + e), lambda i: (0, 0)),
            pl.BlockSpec((1, e), lambda i: (0, 0)),
            pl.BlockSpec((e, e * r), lambda i: (0, 0)),
            pl.BlockSpec((e * r, d), lambda i: (0, 0)),
        ],
        out_specs=pl.BlockSpec((tb, d), lambda i: (i, 0)),
        out_shape=jax.ShapeDtypeStruct((tokens, d), jnp.float32),
        compiler_params=pltpu.CompilerParams(
            dimension_semantics=("parallel",),
        ),
    )(x2, w_fused, bias, expand, b_cat)
    return out.reshape(b, s, d)


# speedup vs baseline: 1.1453x; 1.1453x over previous
import jax
import jax.numpy as jnp
from jax.experimental import pallas as pl
from jax.experimental.pallas import tpu as pltpu

_B, _S, _D, _E, _R = 4, 2048, 2048, 8, 4
_ALPHA = 16.0
_ER = _E * _R
_TOKENS_PER_BLOCK = 1024


def _molora_block(x_ref, w_ref, b_ref, exp_ref, bcat_ref, out_ref):
    y = jnp.dot(x_ref[...], w_ref[...], preferred_element_type=jnp.float32)
    ax = y[:, :_ER]
    logits = y[:, _ER:_ER + _E] + b_ref[...]
    m = jnp.max(logits, axis=-1, keepdims=True)
    e = jnp.exp(logits - m)
    probs = e / jnp.sum(e, axis=-1, keepdims=True)
    probs_er = jnp.dot(probs, exp_ref[...], preferred_element_type=jnp.float32)
    out_ref[...] = jnp.dot(ax * probs_er, bcat_ref[...],
                           preferred_element_type=jnp.float32)


def kernel(x, lora_A, lora_B, router_w, router_b):
    b, s, d = x.shape
    e, _, r = lora_A.shape
    tokens = b * s
    tb = _TOKENS_PER_BLOCK
    grid = tokens // tb

    x2 = x.reshape(tokens, d)
    a_cat = lora_A.transpose(1, 0, 2).reshape(d, e * r)
    w_fused = jnp.concatenate([a_cat, router_w], axis=1)
    b_cat = lora_B.reshape(e * r, d) * (_ALPHA / r)
    expand = jnp.repeat(jnp.eye(e, dtype=jnp.float32), r, axis=1)
    bias = router_b.reshape(1, e)

    out = pl.pallas_call(
        _molora_block,
        grid=(grid,),
        in_specs=[
            pl.BlockSpec((tb, d), lambda i: (i, 0)),
            pl.BlockSpec((d, e * r + e), lambda i: (0, 0)),
            pl.BlockSpec((1, e), lambda i: (0, 0)),
            pl.BlockSpec((e, e * r), lambda i: (0, 0)),
            pl.BlockSpec((e * r, d), lambda i: (0, 0)),
        ],
        out_specs=pl.BlockSpec((tb, d), lambda i: (i, 0)),
        out_shape=jax.ShapeDtypeStruct((tokens, d), jnp.float32),
        compiler_params=pltpu.CompilerParams(
            dimension_semantics=("parallel",),
            vmem_limit_bytes=60 * 1024 * 1024,
        ),
    )(x2, w_fused, bias, expand, b_cat)
    return out.reshape(b, s, d)
